# initial kernel scaffold (unmeasured)
import jax
import jax.numpy as jnp
from jax import lax
from jax.experimental import pallas as pl
from jax.experimental.pallas import tpu as pltpu


def kernel(
    x,
):
    def body(*refs):
        pass

    out_shape = jax.ShapeDtypeStruct(..., jnp.float32)
    return pl.pallas_call(body, out_shape=out_shape)(...)


# baseline (device time: 589951 ns/iter reference)
import jax
from jax import lax
from jax.experimental import pallas as pl
from jax.experimental.pallas import tpu as pltpu

NZ = 4


def kernel(x):
    m, n = x.shape
    blk = n // NZ

    def body(x_ref, out_ref, copy_sem, send_sems, recv_sems):
        my_x = lax.axis_index("x")
        my_y = lax.axis_index("y")
        my_z = lax.axis_index("z")

        barrier = pltpu.get_barrier_semaphore()
        for d in range(1, NZ):
            peer = (my_z + d) % NZ
            pl.semaphore_signal(
                barrier, inc=1,
                device_id=(my_x, my_y, peer),
                device_id_type=pl.DeviceIdType.MESH,
            )
        pl.semaphore_wait(barrier, NZ - 1)

        local = pltpu.make_async_copy(
            x_ref.at[:, pl.ds(my_z * blk, blk)],
            out_ref.at[pl.ds(my_z * m, m), :],
            copy_sem,
        )
        local.start()

        sends = []
        for d in range(1, NZ):
            peer = (my_z + d) % NZ
            rdma = pltpu.make_async_remote_copy(
                src_ref=x_ref.at[:, pl.ds(peer * blk, blk)],
                dst_ref=out_ref.at[pl.ds(my_z * m, m), :],
                send_sem=send_sems.at[d - 1],
                recv_sem=recv_sems.at[d - 1],
                device_id=(my_x, my_y, peer),
                device_id_type=pl.DeviceIdType.MESH,
            )
            rdma.start()
            sends.append(rdma)

        local.wait()
        for rdma in sends:
            rdma.wait_send()

        for d in range(1, NZ):
            src_z = (my_z - d) % NZ
            recv = pltpu.make_async_remote_copy(
                src_ref=x_ref.at[:, pl.ds(my_z * blk, blk)],
                dst_ref=out_ref.at[pl.ds(src_z * m, m), :],
                send_sem=send_sems.at[d - 1],
                recv_sem=recv_sems.at[d - 1],
                device_id=(my_x, my_y, src_z),
                device_id_type=pl.DeviceIdType.MESH,
            )
            recv.wait_recv()

    return pl.pallas_call(
        body,
        out_shape=jax.ShapeDtypeStruct((NZ * m, blk), x.dtype),
        in_specs=[pl.BlockSpec(memory_space=pl.ANY)],
        out_specs=pl.BlockSpec(memory_space=pl.ANY),
        scratch_shapes=[
            pltpu.SemaphoreType.DMA,
            pltpu.SemaphoreType.DMA((NZ - 1,)),
            pltpu.SemaphoreType.DMA((NZ - 1,)),
        ],
        compiler_params=pltpu.CompilerParams(collective_id=0),
    )(x)
